# Initial kernel scaffold; baseline (speedup 1.0000x reference)
import jax
import jax.numpy as jnp
from jax.experimental import pallas as pl


def kernel(x, hp_bank, stat_bank, power_bank, damage_bank, turn_bank, rating_bank, group_idx):
    raise NotImplementedError("write your pallas kernel here")


# speedup vs baseline: 9.2165x; 9.2165x over previous
import functools

import jax
import jax.numpy as jnp
from jax import lax
from jax.experimental import pallas as pl
from jax.experimental.pallas import tpu as pltpu
from jax.experimental.pallas import tpu_sc as plsc

_GROUPS = [(0, 'hp'), (1, 'stat'), (2, 'stat'), (3, 'stat'), (4, 'stat'),
           (5, 'stat'), (6, 'stat'), (7, 'power'), (8, 'power'), (9, 'power'),
           (10, 'power'), (11, 'damage'), (12, 'damage'), (13, 'damage'),
           (14, 'damage'), (15, 'turn'), (16, 'rating'), (17, 'rating')]
_CFG = {'hp': (1.0, 100, 16), 'stat': (600.0, 600, 16), 'power': (250.0, 250, 16),
        'damage': (600.0, 600, 4), 'turn': (40.0, 40, 16), 'rating': (2000.0, 100, 16)}
_BANK_ORDER = ['hp', 'stat', 'power', 'damage', 'turn', 'rating']

_BASES = {}
_off = 0
for _name in _BANK_ORDER:
    _BASES[_name] = _off
    _maxv, _nbins, _w = _CFG[_name]
    _off += (_nbins + 1) * _w
_BANK_WORDS = _off
_BANK_PAD = (-_BANK_WORDS) % 16
_BANK_TOTAL = _BANK_WORDS + _BANK_PAD

_FEATS = []
for _xcol, _name in _GROUPS:
    _maxv, _nbins, _w = _CFG[_name]
    _FEATS.append((_xcol, _BASES[_name], _w, _maxv, _nbins))

_COLPLAN = []
for _fi, (_xcol, _b, _w, _mv, _nb) in enumerate(_FEATS):
    for _o in range(_w):
        _COLPLAN.append((_fi, _o))
_N_EMB = len(_COLPLAN)

_D_IN = 64
_D_OUT = _N_EMB + (_D_IN - len(_FEATS))
_NC, _NS = 2, 16
_NW = _NC * _NS
_CH = 256
_L = 16


def _tec_body(x_hbm, banks_hbm, out_hbm, xv, banksv, outv, *, n_pos):
    p_per = n_pos // _NW
    n_ch = p_per // _CH
    wid = lax.axis_index("s") * _NC + lax.axis_index("c")
    base = wid * p_per

    pltpu.sync_copy(banks_hbm, banksv)

    iota = lax.iota(jnp.int32, _L)
    iota_in = iota * _D_IN
    iota_out = iota * _D_OUT

    def chunk_body(g, carry):
        row0 = base + g * _CH
        pltpu.sync_copy(x_hbm.at[pl.ds(row0 * _D_IN, _CH * _D_IN)], xv)

        def group_body(t, c2):
            pb = iota_in + t * (_L * _D_IN)
            ob = iota_out + t * (_L * _D_OUT)
            rowaddr = []
            for (xcol, bank_base, w, maxv, nbins) in _FEATS:
                raw = plsc.load_gather(xv, [pb + xcol])
                clamped = jnp.clip(raw, 0.0, maxv)
                b = ((clamped / maxv) * nbins).astype(jnp.int32)
                b = jnp.clip(b, 0, nbins)
                rowaddr.append(bank_base + b * w)
            for col in range(_D_OUT):
                if col < _N_EMB:
                    fi, off = _COLPLAN[col]
                    val = plsc.load_gather(banksv, [rowaddr[fi] + off])
                else:
                    val = plsc.load_gather(xv, [pb + (col - _N_EMB + len(_FEATS))])
                plsc.store_scatter(outv, [ob + col], val)
            return c2

        lax.fori_loop(0, _CH // _L, group_body, 0)
        pltpu.sync_copy(outv, out_hbm.at[pl.ds(row0 * _D_OUT, _CH * _D_OUT)])
        return carry

    lax.fori_loop(0, n_ch, chunk_body, 0)


def kernel(x, hp_bank, stat_bank, power_bank, damage_bank, turn_bank,
           rating_bank, group_idx):
    bsz, seq, d_in = x.shape
    n_pos = bsz * seq
    banks_flat = jnp.concatenate([
        hp_bank.reshape(-1), stat_bank.reshape(-1), power_bank.reshape(-1),
        damage_bank.reshape(-1), turn_bank.reshape(-1), rating_bank.reshape(-1),
        jnp.zeros((_BANK_PAD,), jnp.float32)])

    mesh = plsc.VectorSubcoreMesh(core_axis_name="c", subcore_axis_name="s")
    run = functools.partial(
        pl.kernel,
        mesh=mesh,
        compiler_params=pltpu.CompilerParams(needs_layout_passes=False),
        out_type=jax.ShapeDtypeStruct((n_pos * _D_OUT,), jnp.float32),
        scratch_types=[
            pltpu.VMEM((_CH * _D_IN,), jnp.float32),
            pltpu.VMEM((_BANK_TOTAL,), jnp.float32),
            pltpu.VMEM((_CH * _D_OUT,), jnp.float32),
        ],
    )(functools.partial(_tec_body, n_pos=n_pos))
    out_flat = run(x.reshape(-1), banks_flat)
    return out_flat.reshape(bsz, seq, _D_OUT)


# speedup vs baseline: 11.6728x; 1.2665x over previous
import functools

import jax
import jax.numpy as jnp
from jax import lax
from jax.experimental import pallas as pl
from jax.experimental.pallas import tpu as pltpu
from jax.experimental.pallas import tpu_sc as plsc

_GROUPS = [(0, 'hp'), (1, 'stat'), (2, 'stat'), (3, 'stat'), (4, 'stat'),
           (5, 'stat'), (6, 'stat'), (7, 'power'), (8, 'power'), (9, 'power'),
           (10, 'power'), (11, 'damage'), (12, 'damage'), (13, 'damage'),
           (14, 'damage'), (15, 'turn'), (16, 'rating'), (17, 'rating')]
_CFG = {'hp': (1.0, 100, 16), 'stat': (600.0, 600, 16), 'power': (250.0, 250, 16),
        'damage': (600.0, 600, 4), 'turn': (40.0, 40, 16), 'rating': (2000.0, 100, 16)}
_BANK_ORDER = ['hp', 'stat', 'power', 'damage', 'turn', 'rating']

_BASES = {}
_off = 0
for _name in _BANK_ORDER:
    _BASES[_name] = _off
    _maxv, _nbins, _w = _CFG[_name]
    _off += (_nbins + 1) * _w
_BANK_WORDS = _off
_BANK_PAD = (-_BANK_WORDS) % 16
_BANK_TOTAL = _BANK_WORDS + _BANK_PAD

_FEATS = []
for _xcol, _name in _GROUPS:
    _maxv, _nbins, _w = _CFG[_name]
    _FEATS.append((_xcol, _BASES[_name], _w, _maxv, _nbins))

_COLPLAN = []
for _fi, (_xcol, _b, _w, _mv, _nb) in enumerate(_FEATS):
    for _o in range(_w):
        _COLPLAN.append((_fi, _o))
_N_EMB = len(_COLPLAN)

_D_IN = 64
_D_OUT = _N_EMB + (_D_IN - len(_FEATS))
_NC, _NS = 2, 16
_NW = _NC * _NS
_CH = 256
_L = 16


def _tec_body(x_hbm, banks_hbm, out_hbm, xv, banksv, outv, *, n_pos):
    p_per = n_pos // _NW
    n_ch = p_per // _CH
    wid = lax.axis_index("s") * _NC + lax.axis_index("c")
    base = wid * p_per

    pltpu.sync_copy(banks_hbm, banksv)

    iota = lax.iota(jnp.int32, _L)
    iota_in = iota * _D_IN
    iota_out = iota * _D_OUT

    def chunk_body(g, carry):
        row0 = base + g * _CH
        pltpu.sync_copy(x_hbm.at[pl.ds(row0 * _D_IN, _CH * _D_IN)], xv)

        @plsc.parallel_loop(0, _CH // _L)
        def group_body(t):
            pb = iota_in + t * (_L * _D_IN)
            ob = iota_out + t * (_L * _D_OUT)
            rowaddr = []
            for (xcol, bank_base, w, maxv, nbins) in _FEATS:
                raw = plsc.load_gather(xv, [pb + xcol])
                clamped = jnp.clip(raw, 0.0, maxv)
                b = ((clamped / maxv) * nbins).astype(jnp.int32)
                b = jnp.clip(b, 0, nbins)
                rowaddr.append(bank_base + b * w)

            def col_val(col):
                if col < _N_EMB:
                    fi, off = _COLPLAN[col]
                    return plsc.load_gather(banksv, [rowaddr[fi] + off])
                return plsc.load_gather(xv, [pb + (col - _N_EMB + len(_FEATS))])

            _K = 8
            for i in range(0, _D_OUT, _K):
                batch = range(i, min(i + _K, _D_OUT))
                vals = [col_val(col) for col in batch]
                for col, val in zip(batch, vals):
                    plsc.store_scatter(outv, [ob + col], val)
        pltpu.sync_copy(outv, out_hbm.at[pl.ds(row0 * _D_OUT, _CH * _D_OUT)])
        return carry

    lax.fori_loop(0, n_ch, chunk_body, 0)


def kernel(x, hp_bank, stat_bank, power_bank, damage_bank, turn_bank,
           rating_bank, group_idx):
    bsz, seq, d_in = x.shape
    n_pos = bsz * seq
    banks_flat = jnp.concatenate([
        hp_bank.reshape(-1), stat_bank.reshape(-1), power_bank.reshape(-1),
        damage_bank.reshape(-1), turn_bank.reshape(-1), rating_bank.reshape(-1),
        jnp.zeros((_BANK_PAD,), jnp.float32)])

    mesh = plsc.VectorSubcoreMesh(core_axis_name="c", subcore_axis_name="s")
    run = functools.partial(
        pl.kernel,
        mesh=mesh,
        compiler_params=pltpu.CompilerParams(needs_layout_passes=False),
        out_type=jax.ShapeDtypeStruct((n_pos * _D_OUT,), jnp.float32),
        scratch_types=[
            pltpu.VMEM((_CH * _D_IN,), jnp.float32),
            pltpu.VMEM((_BANK_TOTAL,), jnp.float32),
            pltpu.VMEM((_CH * _D_OUT,), jnp.float32),
        ],
    )(functools.partial(_tec_body, n_pos=n_pos))
    out_flat = run(x.reshape(-1), banks_flat)
    return out_flat.reshape(bsz, seq, _D_OUT)


# speedup vs baseline: 11.8195x; 1.0126x over previous
import functools

import jax
import jax.numpy as jnp
from jax import lax
from jax.experimental import pallas as pl
from jax.experimental.pallas import tpu as pltpu
from jax.experimental.pallas import tpu_sc as plsc

_GROUPS = [(0, 'hp'), (1, 'stat'), (2, 'stat'), (3, 'stat'), (4, 'stat'),
           (5, 'stat'), (6, 'stat'), (7, 'power'), (8, 'power'), (9, 'power'),
           (10, 'power'), (11, 'damage'), (12, 'damage'), (13, 'damage'),
           (14, 'damage'), (15, 'turn'), (16, 'rating'), (17, 'rating')]
_CFG = {'hp': (1.0, 100, 16), 'stat': (600.0, 600, 16), 'power': (250.0, 250, 16),
        'damage': (600.0, 600, 4), 'turn': (40.0, 40, 16), 'rating': (2000.0, 100, 16)}
_BANK_ORDER = ['hp', 'stat', 'power', 'damage', 'turn', 'rating']

_BASES = {}
_off = 0
for _name in _BANK_ORDER:
    _BASES[_name] = _off
    _maxv, _nbins, _w = _CFG[_name]
    _off += (_nbins + 1) * (_w + 1)
_BANK_WORDS = _off
_BANK_PAD = (-_BANK_WORDS) % 16
_BANK_TOTAL = _BANK_WORDS + _BANK_PAD

_FEATS = []
for _xcol, _name in _GROUPS:
    _maxv, _nbins, _w = _CFG[_name]
    _FEATS.append((_xcol, _BASES[_name], _w + 1, _w, _maxv, _nbins))

_COLPLAN = []
for _fi, (_xcol, _b, _stride, _w, _mv, _nb) in enumerate(_FEATS):
    for _o in range(_w):
        _COLPLAN.append((_fi, _o))
_N_EMB = len(_COLPLAN)

_D_IN = 64
_D_OUT = _N_EMB + (_D_IN - len(_FEATS))
_NC, _NS = 2, 16
_NW = _NC * _NS
_CH = 256
_L = 16


def _tec_body(x_hbm, banks_hbm, out_hbm, xv, banksv, outv, *, n_pos):
    p_per = n_pos // _NW
    n_ch = p_per // _CH
    wid = lax.axis_index("s") * _NC + lax.axis_index("c")
    base = wid * p_per

    pltpu.sync_copy(banks_hbm, banksv)

    iota = lax.iota(jnp.int32, _L)
    iota_in = iota * _D_IN
    iota_out = iota * _D_OUT

    def chunk_body(g, carry):
        row0 = base + g * _CH
        pltpu.sync_copy(x_hbm.at[pl.ds(row0 * _D_IN, _CH * _D_IN)], xv)

        @plsc.parallel_loop(0, _CH // _L)
        def group_body(t):
            pb = iota_in + t * (_L * _D_IN)
            ob = iota_out + t * (_L * _D_OUT)
            rowaddr = []
            for (xcol, bank_base, stride, w, maxv, nbins) in _FEATS:
                raw = plsc.load_gather(xv, [pb + xcol])
                clamped = jnp.clip(raw, 0.0, maxv)
                b = ((clamped / maxv) * nbins).astype(jnp.int32)
                b = jnp.clip(b, 0, nbins)
                rowaddr.append(bank_base + b * stride)

            def col_val(col):
                if col < _N_EMB:
                    fi, off = _COLPLAN[col]
                    return plsc.load_gather(banksv, [rowaddr[fi] + off])
                return plsc.load_gather(xv, [pb + (col - _N_EMB + len(_FEATS))])

            _K = 8
            for i in range(0, _D_OUT, _K):
                batch = range(i, min(i + _K, _D_OUT))
                vals = [col_val(col) for col in batch]
                for col, val in zip(batch, vals):
                    plsc.store_scatter(outv, [ob + col], val)
        pltpu.sync_copy(outv, out_hbm.at[pl.ds(row0 * _D_OUT, _CH * _D_OUT)])
        return carry

    lax.fori_loop(0, n_ch, chunk_body, 0)


def kernel(x, hp_bank, stat_bank, power_bank, damage_bank, turn_bank,
           rating_bank, group_idx):
    bsz, seq, d_in = x.shape
    n_pos = bsz * seq
    def _padrow(b):
        return jnp.pad(b, ((0, 0), (0, 1))).reshape(-1)

    banks_flat = jnp.concatenate([
        _padrow(hp_bank), _padrow(stat_bank), _padrow(power_bank),
        _padrow(damage_bank), _padrow(turn_bank), _padrow(rating_bank),
        jnp.zeros((_BANK_PAD,), jnp.float32)])

    mesh = plsc.VectorSubcoreMesh(core_axis_name="c", subcore_axis_name="s")
    run = functools.partial(
        pl.kernel,
        mesh=mesh,
        compiler_params=pltpu.CompilerParams(needs_layout_passes=False),
        out_type=jax.ShapeDtypeStruct((n_pos * _D_OUT,), jnp.float32),
        scratch_types=[
            pltpu.VMEM((_CH * _D_IN,), jnp.float32),
            pltpu.VMEM((_BANK_TOTAL,), jnp.float32),
            pltpu.VMEM((_CH * _D_OUT,), jnp.float32),
        ],
    )(functools.partial(_tec_body, n_pos=n_pos))
    out_flat = run(x.reshape(-1), banks_flat)
    return out_flat.reshape(bsz, seq, _D_OUT)
